# speedup vs baseline: 1.0327x; 1.0327x over previous
import jax
import jax.numpy as jnp
from jax.experimental import pallas as pl
from jax.experimental.pallas import tpu as pltpu

N = 10000
D_IN = 128
D_OUT = 128
EPS = 1e-5
BM = 400


def _gcn_kernel(x_ref, adj_ref, w_ref, scale_ref, shift_ref, out_ref,
                support_ref):
    @pl.when(pl.program_id(0) == 0)
    def _():
        s = jnp.dot(x_ref[...], w_ref[...],
                    preferred_element_type=jnp.float32)
        support_ref[...] = s.astype(jnp.bfloat16)

    acc = jnp.dot(adj_ref[...].astype(jnp.bfloat16), support_ref[...],
                  preferred_element_type=jnp.float32)
    out_ref[...] = jnp.maximum(acc * scale_ref[...] + shift_ref[...], 0.0)


def kernel(x, adj, W, b, bn_gamma, bn_beta):
    scale = (bn_gamma / jnp.sqrt(1.0 + EPS)).reshape(1, D_OUT)
    shift = (b * scale[0] + bn_beta).reshape(1, D_OUT)

    grid = (N // BM,)
    return pl.pallas_call(
        _gcn_kernel,
        grid=grid,
        in_specs=[
            pl.BlockSpec((N, D_IN), lambda i: (0, 0)),
            pl.BlockSpec((BM, N), lambda i: (i, 0)),
            pl.BlockSpec((D_IN, D_OUT), lambda i: (0, 0)),
            pl.BlockSpec((1, D_OUT), lambda i: (0, 0)),
            pl.BlockSpec((1, D_OUT), lambda i: (0, 0)),
        ],
        out_specs=pl.BlockSpec((BM, D_OUT), lambda i: (i, 0)),
        out_shape=jax.ShapeDtypeStruct((N, D_OUT), jnp.float32),
        scratch_shapes=[pltpu.VMEM((N, D_OUT), jnp.bfloat16)],
        compiler_params=pltpu.CompilerParams(
            dimension_semantics=("arbitrary",),
        ),
    )(x, adj, W, scale, shift)
